# baseline (speedup 1.0000x reference)
import functools

import jax
import jax.numpy as jnp
from jax import lax
from jax.experimental import pallas as pl
from jax.experimental.pallas import tpu as pltpu
from jax.experimental.pallas import tpu_sc as plsc

_EMB = 8
_DIM = 4
_L = 16


@functools.cache
def _make_lookup(n_ids: int):
    mesh = plsc.VectorSubcoreMesh(core_axis_name="c", subcore_axis_name="s")
    n_workers = mesh.num_cores * mesh.num_subcores
    nc = mesh.num_cores
    ids_per_w = n_ids // n_workers
    out_per_w = ids_per_w * _DIM
    n_iter = ids_per_w // _L
    assert ids_per_w * n_workers == n_ids and n_iter * _L == ids_per_w

    @functools.partial(
        pl.kernel,
        out_type=jax.ShapeDtypeStruct((n_ids * _DIM,), jnp.float32),
        mesh=mesh,
        scratch_types=[
            pltpu.VMEM((ids_per_w,), jnp.int32),
            pltpu.VMEM((_EMB * _DIM,), jnp.float32),
            pltpu.VMEM((out_per_w,), jnp.float32),
            pltpu.SemaphoreType.DMA,
            pltpu.SemaphoreType.DMA,
            pltpu.SemaphoreType.DMA,
        ],
        compiler_params=pltpu.CompilerParams(needs_layout_passes=False),
    )
    def lookup(ids_hbm, table_hbm, out_hbm, ids_v, tab_v, out_v, sem_i, sem_t, sem_o):
        wid = lax.axis_index("s") * nc + lax.axis_index("c")
        base = wid * ids_per_w
        half = n_iter // 2
        half_out = half * _L * _DIM
        ids_cp = pltpu.async_copy(ids_hbm.at[pl.ds(base, ids_per_w)], ids_v, sem_i)
        tab_cp = pltpu.async_copy(table_hbm, tab_v, sem_t)
        tab_cp.wait()
        tab_v[pl.ds(0, _L)] = tab_v[pl.ds(0, _L)] + 1.0
        tab_v[pl.ds(_L, _L)] = tab_v[pl.ds(_L, _L)] + 1.0
        ids_cp.wait()

        lane4 = lax.iota(jnp.int32, _L) * _DIM

        @plsc.parallel_loop(0, half, unroll=8)
        def _(i):
            ids16 = ids_v[pl.ds(i * _L, _L)]
            gidx = ids16 * _DIM
            st_base = lane4 + i * (_L * _DIM)
            for d in range(_DIM):
                vals = plsc.load_gather(tab_v, [gidx + d])
                plsc.store_scatter(out_v, [st_base + d], vals)

        out_cp0 = pltpu.async_copy(
            out_v.at[pl.ds(0, half_out)],
            out_hbm.at[pl.ds(base * _DIM, half_out)],
            sem_o,
        )

        @plsc.parallel_loop(half, n_iter, unroll=8)
        def _(i):
            ids16 = ids_v[pl.ds(i * _L, _L)]
            gidx = ids16 * _DIM
            st_base = lane4 + i * (_L * _DIM)
            for d in range(_DIM):
                vals = plsc.load_gather(tab_v, [gidx + d])
                plsc.store_scatter(out_v, [st_base + d], vals)

        pltpu.sync_copy(
            out_v.at[pl.ds(half_out, out_per_w - half_out)],
            out_hbm.at[pl.ds(base * _DIM + half_out, out_per_w - half_out)],
        )
        out_cp0.wait()

    return lookup


def kernel(input_ids, embed_weight):
    ids_flat = input_ids.reshape(-1).astype(jnp.int32)
    table_flat = embed_weight.reshape(-1)
    out_flat = _make_lookup(ids_flat.shape[0])(ids_flat, table_flat)
    return out_flat.reshape(*input_ids.shape, _DIM)


# speedup vs baseline: 1.0301x; 1.0301x over previous
import functools

import jax
import jax.numpy as jnp
from jax import lax
from jax.experimental import pallas as pl
from jax.experimental.pallas import tpu as pltpu
from jax.experimental.pallas import tpu_sc as plsc

_EMB = 8
_DIM = 4
_L = 16


@functools.cache
def _make_lookup(n_ids: int):
    mesh = plsc.VectorSubcoreMesh(core_axis_name="c", subcore_axis_name="s", num_cores=1, num_subcores=8)
    n_workers = mesh.num_cores * mesh.num_subcores
    nc = mesh.num_cores
    ids_per_w = n_ids // n_workers
    out_per_w = ids_per_w * _DIM
    n_iter = ids_per_w // _L
    assert ids_per_w * n_workers == n_ids and n_iter * _L == ids_per_w

    @functools.partial(
        pl.kernel,
        out_type=jax.ShapeDtypeStruct((n_ids * _DIM,), jnp.float32),
        mesh=mesh,
        scratch_types=[
            pltpu.VMEM((ids_per_w,), jnp.int32),
            pltpu.VMEM((_EMB * _DIM,), jnp.float32),
            pltpu.VMEM((out_per_w,), jnp.float32),
            pltpu.SemaphoreType.DMA,
            pltpu.SemaphoreType.DMA,
            pltpu.SemaphoreType.DMA,
        ],
        compiler_params=pltpu.CompilerParams(needs_layout_passes=False),
    )
    def lookup(ids_hbm, table_hbm, out_hbm, ids_v, tab_v, out_v, sem_i, sem_t, sem_o):
        wid = lax.axis_index("s") * nc + lax.axis_index("c")
        base = wid * ids_per_w
        half = n_iter // 2
        half_out = half * _L * _DIM
        ids_cp = pltpu.async_copy(ids_hbm.at[pl.ds(base, ids_per_w)], ids_v, sem_i)
        tab_cp = pltpu.async_copy(table_hbm, tab_v, sem_t)
        tab_cp.wait()
        tab_v[pl.ds(0, _L)] = tab_v[pl.ds(0, _L)] + 1.0
        tab_v[pl.ds(_L, _L)] = tab_v[pl.ds(_L, _L)] + 1.0
        ids_cp.wait()

        lane4 = lax.iota(jnp.int32, _L) * _DIM

        @plsc.parallel_loop(0, half, unroll=8)
        def _(i):
            ids16 = ids_v[pl.ds(i * _L, _L)]
            gidx = ids16 * _DIM
            st_base = lane4 + i * (_L * _DIM)
            for d in range(_DIM):
                vals = plsc.load_gather(tab_v, [gidx + d])
                plsc.store_scatter(out_v, [st_base + d], vals)

        out_cp0 = pltpu.async_copy(
            out_v.at[pl.ds(0, half_out)],
            out_hbm.at[pl.ds(base * _DIM, half_out)],
            sem_o,
        )

        @plsc.parallel_loop(half, n_iter, unroll=8)
        def _(i):
            ids16 = ids_v[pl.ds(i * _L, _L)]
            gidx = ids16 * _DIM
            st_base = lane4 + i * (_L * _DIM)
            for d in range(_DIM):
                vals = plsc.load_gather(tab_v, [gidx + d])
                plsc.store_scatter(out_v, [st_base + d], vals)

        pltpu.sync_copy(
            out_v.at[pl.ds(half_out, out_per_w - half_out)],
            out_hbm.at[pl.ds(base * _DIM + half_out, out_per_w - half_out)],
        )
        out_cp0.wait()

    return lookup


def kernel(input_ids, embed_weight):
    ids_flat = input_ids.reshape(-1).astype(jnp.int32)
    table_flat = embed_weight.reshape(-1)
    out_flat = _make_lookup(ids_flat.shape[0])(ids_flat, table_flat)
    return out_flat.reshape(*input_ids.shape, _DIM)


# speedup vs baseline: 1.0411x; 1.0106x over previous
import functools

import jax
import jax.numpy as jnp
from jax import lax
from jax.experimental import pallas as pl
from jax.experimental.pallas import tpu as pltpu
from jax.experimental.pallas import tpu_sc as plsc

_EMB = 8
_DIM = 4
_L = 16


@functools.cache
def _make_lookup(n_ids: int):
    mesh = plsc.VectorSubcoreMesh(core_axis_name="c", subcore_axis_name="s", num_cores=1)
    n_workers = mesh.num_cores * mesh.num_subcores
    nc = mesh.num_cores
    ids_per_w = n_ids // n_workers
    out_per_w = ids_per_w * _DIM
    n_iter = ids_per_w // _L
    assert ids_per_w * n_workers == n_ids and n_iter * _L == ids_per_w

    @functools.partial(
        pl.kernel,
        out_type=jax.ShapeDtypeStruct((n_ids * _DIM,), jnp.float32),
        mesh=mesh,
        scratch_types=[
            pltpu.VMEM((ids_per_w,), jnp.int32),
            pltpu.VMEM((_EMB * _DIM,), jnp.float32),
            pltpu.VMEM((out_per_w,), jnp.float32),
            pltpu.SemaphoreType.DMA,
            pltpu.SemaphoreType.DMA,
            pltpu.SemaphoreType.DMA,
        ],
        compiler_params=pltpu.CompilerParams(needs_layout_passes=False),
    )
    def lookup(ids_hbm, table_hbm, out_hbm, ids_v, tab_v, out_v, sem_i, sem_t, sem_o):
        wid = lax.axis_index("s") * nc + lax.axis_index("c")
        base = wid * ids_per_w
        half = n_iter // 2
        half_out = half * _L * _DIM
        ids_cp = pltpu.async_copy(ids_hbm.at[pl.ds(base, ids_per_w)], ids_v, sem_i)
        tab_cp = pltpu.async_copy(table_hbm, tab_v, sem_t)
        tab_cp.wait()
        tab_v[pl.ds(0, _L)] = tab_v[pl.ds(0, _L)] + 1.0
        tab_v[pl.ds(_L, _L)] = tab_v[pl.ds(_L, _L)] + 1.0
        ids_cp.wait()

        lane4 = lax.iota(jnp.int32, _L) * _DIM

        @plsc.parallel_loop(0, half, unroll=8)
        def _(i):
            ids16 = ids_v[pl.ds(i * _L, _L)]
            gidx = ids16 * _DIM
            st_base = lane4 + i * (_L * _DIM)
            for d in range(_DIM):
                vals = plsc.load_gather(tab_v, [gidx + d])
                plsc.store_scatter(out_v, [st_base + d], vals)

        out_cp0 = pltpu.async_copy(
            out_v.at[pl.ds(0, half_out)],
            out_hbm.at[pl.ds(base * _DIM, half_out)],
            sem_o,
        )

        @plsc.parallel_loop(half, n_iter, unroll=8)
        def _(i):
            ids16 = ids_v[pl.ds(i * _L, _L)]
            gidx = ids16 * _DIM
            st_base = lane4 + i * (_L * _DIM)
            for d in range(_DIM):
                vals = plsc.load_gather(tab_v, [gidx + d])
                plsc.store_scatter(out_v, [st_base + d], vals)

        pltpu.sync_copy(
            out_v.at[pl.ds(half_out, out_per_w - half_out)],
            out_hbm.at[pl.ds(base * _DIM + half_out, out_per_w - half_out)],
        )
        out_cp0.wait()

    return lookup


def kernel(input_ids, embed_weight):
    ids_flat = input_ids.reshape(-1).astype(jnp.int32)
    table_flat = embed_weight.reshape(-1)
    out_flat = _make_lookup(ids_flat.shape[0])(ids_flat, table_flat)
    return out_flat.reshape(*input_ids.shape, _DIM)
